# baseline (speedup 1.0000x reference)
import functools

import jax
import jax.numpy as jnp
from jax import lax
from jax.experimental import pallas as pl
from jax.experimental.pallas import tpu as pltpu
from jax.experimental.pallas import tpu_sc as plsc

NUM_CORES = 2
NUM_SUBCORES = 16
NUM_WORKERS = NUM_CORES * NUM_SUBCORES
BLOCK = 128
NBUF = 4


def _gather_body(idx_hbm, table_hbm, out_hbm, rows_refs, sems, idx_all):
    n_chunks = idx_hbm.shape[1]
    wid = lax.axis_index("s") * NUM_CORES + lax.axis_index("c")
    nbatch = out_hbm.shape[0] // n_chunks
    obase = wid * BLOCK
    gsem, osem = sems

    pltpu.sync_copy(idx_hbm.at[wid], idx_all)

    def gather_start(i, b):
        pltpu.async_copy(table_hbm.at[idx_all.at[i]], rows_refs[b], gsem[b])

    def gather_wait(b):
        pltpu.make_async_copy(table_hbm.at[idx_all.at[0]], rows_refs[b],
                              gsem[b]).wait()

    def out_start(i, b):
        pltpu.async_copy(
            rows_refs[b], out_hbm.at[pl.ds(i * nbatch + obase, BLOCK)],
            osem[b])

    def out_wait(b):
        pltpu.make_async_copy(
            rows_refs[b], out_hbm.at[pl.ds(obase, BLOCK)], osem[b]).wait()

    gather_start(0, 0)
    gather_start(1, 1)
    for i in (2, 3):
        gather_start(i, i)
        gather_wait(i - 2)
        out_start(i - 2, i - 2)

    def quad(k, _):
        for t in range(NBUF):
            i = NBUF + NBUF * k + t
            out_wait(t)
            gather_start(i, t)
            gather_wait((t + 2) % NBUF)
            out_start(i - 2, (t + 2) % NBUF)
        return ()

    n_steady = n_chunks - NBUF
    lax.fori_loop(0, n_steady // NBUF, quad, (), unroll=False)
    for i in range(n_chunks - n_steady % NBUF, n_chunks):
        t = i % NBUF
        out_wait(t)
        gather_start(i, t)
        gather_wait((t + 2) % NBUF)
        out_start(i - 2, (t + 2) % NBUF)

    for i in (n_chunks - 2, n_chunks - 1):
        b = i % NBUF
        gather_wait(b)
        out_start(i, b)
    for i in range(n_chunks - NBUF, n_chunks):
        out_wait(i % NBUF)


def kernel(indices, weight):
    b, s = indices.shape
    v, d = weight.shape
    n = b * s
    idx_perm = (indices.astype(jnp.int32)
                .reshape(NUM_WORKERS, BLOCK, s)
                .transpose(0, 2, 1))

    mesh = plsc.VectorSubcoreMesh(
        core_axis_name="c", subcore_axis_name="s",
        num_cores=NUM_CORES, num_subcores=NUM_SUBCORES,
    )
    run = functools.partial(
        pl.kernel,
        out_type=jax.ShapeDtypeStruct((n, d), jnp.float32),
        mesh=mesh,
        scratch_types=[
            tuple(pltpu.VMEM((BLOCK, d), jnp.float32) for _ in range(NBUF)),
            (tuple(pltpu.SemaphoreType.DMA for _ in range(NBUF)),
             tuple(pltpu.SemaphoreType.DMA for _ in range(NBUF))),
            pltpu.VMEM((s, BLOCK), jnp.int32),
        ],
    )(_gather_body)
    out = run(idx_perm, weight)
    return out.reshape(s, b, d).transpose(1, 0, 2)


# speedup vs baseline: 1.0045x; 1.0045x over previous
import functools

import jax
import jax.numpy as jnp
from jax import lax
from jax.experimental import pallas as pl
from jax.experimental.pallas import tpu as pltpu
from jax.experimental.pallas import tpu_sc as plsc

NUM_CORES = 2
NUM_SUBCORES = 16
NUM_WORKERS = NUM_CORES * NUM_SUBCORES
STREAM = 128
WBATCH = 256
NBLK = 16
NHALF = 2


def _gather_body(idx_hbm, table_hbm, out_hbm, idx_all, rows0, rows1,
                 g0, g1, o0, o1):
    n_slots = idx_hbm.shape[2] // (WBATCH // STREAM)
    n_seq = n_slots * NHALF
    wid = lax.axis_index("s") * NUM_CORES + lax.axis_index("c")
    p = wid // NHALF
    h = wid % NHALF
    nbatch = out_hbm.shape[0] // n_seq
    obase = h * n_slots * nbatch + p * WBATCH

    rows = (rows0, rows1)
    gsem = (g0, g1)
    osem = (o0, o1)

    pltpu.sync_copy(idx_hbm.at[p, h], idx_all)

    def gathers_start(i, b):
        for j in range(WBATCH // STREAM):
            pltpu.async_copy(
                table_hbm.at[idx_all.at[i * (WBATCH // STREAM) + j]],
                rows[b].at[pl.ds(j * STREAM, STREAM)], gsem[b])

    def gathers_wait(b):
        for j in range(WBATCH // STREAM):
            pltpu.make_async_copy(
                table_hbm.at[idx_all.at[0]],
                rows[b].at[pl.ds(j * STREAM, STREAM)], gsem[b]).wait()

    def out_start(i, b):
        pltpu.async_copy(
            rows[b], out_hbm.at[pl.ds(i * nbatch + obase, WBATCH)], osem[b])

    def out_wait(b):
        pltpu.make_async_copy(
            rows[b], out_hbm.at[pl.ds(obase, WBATCH)], osem[b]).wait()

    gathers_start(0, 0)
    gathers_start(1, 1)
    gathers_wait(0)
    out_start(0, 0)

    def pair(k, _):
        for b in (0, 1):
            i = 2 + 2 * k + b
            out_wait(b)
            gathers_start(i, b)
            gathers_wait(1 - b)
            out_start(i - 1, 1 - b)
        return ()

    n_steady = n_slots - 2
    lax.fori_loop(0, n_steady // 2, pair, (), unroll=False)
    if n_steady % 2:
        i = n_slots - 1
        b = i % 2
        out_wait(b)
        gathers_start(i, b)
        gathers_wait(1 - b)
        out_start(i - 1, 1 - b)

    last = n_slots - 1
    gathers_wait(last % 2)
    out_start(last, last % 2)
    out_wait((last + 1) % 2)
    out_wait(last % 2)


def kernel(indices, weight):
    b, s = indices.shape
    v, d = weight.shape
    n = b * s
    idx_perm = (indices.astype(jnp.int32)
                .reshape(NBLK, WBATCH // STREAM, STREAM, s)
                .transpose(0, 3, 1, 2)
                .reshape(NBLK, NHALF, (s // NHALF) * (WBATCH // STREAM),
                         STREAM))

    mesh = plsc.VectorSubcoreMesh(
        core_axis_name="c", subcore_axis_name="s",
        num_cores=NUM_CORES, num_subcores=NUM_SUBCORES,
    )
    run = functools.partial(
        pl.kernel,
        out_type=jax.ShapeDtypeStruct((n, d), jnp.float32),
        mesh=mesh,
        scratch_types=[
            pltpu.VMEM(((s // NHALF) * (WBATCH // STREAM), STREAM),
                       jnp.int32),
            pltpu.VMEM((WBATCH, d), jnp.float32),
            pltpu.VMEM((WBATCH, d), jnp.float32),
            pltpu.SemaphoreType.DMA,
            pltpu.SemaphoreType.DMA,
            pltpu.SemaphoreType.DMA,
            pltpu.SemaphoreType.DMA,
        ],
    )(_gather_body)
    out = run(idx_perm, weight)
    return out.reshape(s, b, d).transpose(1, 0, 2)
